# Initial kernel scaffold; baseline (speedup 1.0000x reference)
import jax
import jax.numpy as jnp
from jax.experimental import pallas as pl


def kernel(entity_ids, table):
    raise NotImplementedError("write your pallas kernel here")


# speedup vs baseline: 1.8606x; 1.8606x over previous
import functools

import jax
import jax.numpy as jnp
from jax import lax
from jax.experimental import pallas as pl
from jax.experimental.pallas import tpu as pltpu
from jax.experimental.pallas import tpu_sc as plsc

ENTITY_NUM = 1000000
EMBED_DIM = 64
BATCH = 16384
HIST = 50

NUM_CORES = 2
NUM_SUBCORES = 16
NW = NUM_CORES * NUM_SUBCORES

TOTAL = BATCH * HIST
PER_W = TOTAL // NW
CHUNK = 128
CHUNKS_PER_W = PER_W // CHUNK
K = 5
GROUP_ROWS = K * CHUNK
NGROUPS = CHUNKS_PER_W // K

_mesh = plsc.VectorSubcoreMesh(core_axis_name="c", subcore_axis_name="s")


@functools.partial(
    pl.kernel,
    out_type=jax.ShapeDtypeStruct((TOTAL, EMBED_DIM), jnp.float32),
    mesh=_mesh,
    compiler_params=pltpu.CompilerParams(use_tc_tiling_on_sc=False),
    scratch_types=[
        pltpu.VMEM((CHUNKS_PER_W, CHUNK), jnp.int32),
        pltpu.VMEM((GROUP_ROWS, EMBED_DIM), jnp.float32),
        pltpu.VMEM((GROUP_ROWS, EMBED_DIM), jnp.float32),
        pltpu.SemaphoreType.DMA,
        pltpu.SemaphoreType.DMA,
        pltpu.SemaphoreType.DMA,
        pltpu.SemaphoreType.DMA,
    ],
)
def _sc_gather(idx_hbm, table_hbm, out_hbm, idx_v, buf0, buf1, sg0, sg1, so0, so1):
    wid = lax.axis_index("s") * NUM_CORES + lax.axis_index("c")
    base = wid * PER_W

    pltpu.sync_copy(idx_hbm.at[wid], idx_v)

    def fire_gather(g, buf, sem):
        for b in range(K):
            pltpu.async_copy(
                table_hbm.at[idx_v.at[g * K + b]],
                buf.at[pl.ds(b * CHUNK, CHUNK)],
                sem,
            )

    def drain_gather(buf, sem):
        pltpu.make_async_copy(
            table_hbm.at[pl.ds(0, GROUP_ROWS)], buf, sem
        ).wait()

    def fire_out(g, buf, sem):
        pltpu.async_copy(
            buf, out_hbm.at[pl.ds(base + g * GROUP_ROWS, GROUP_ROWS)], sem
        )

    def drain_out(buf, sem):
        pltpu.make_async_copy(
            buf, out_hbm.at[pl.ds(0, GROUP_ROWS)], sem
        ).wait()

    fire_gather(0, buf0, sg0)
    fire_gather(1, buf1, sg1)

    def body(t, carry):
        g = 2 * t
        drain_gather(buf0, sg0)
        fire_out(g, buf0, so0)
        drain_gather(buf1, sg1)
        fire_out(g + 1, buf1, so1)
        drain_out(buf0, so0)
        fire_gather(g + 2, buf0, sg0)
        drain_out(buf1, so1)
        fire_gather(g + 3, buf1, sg1)
        return carry

    lax.fori_loop(0, NGROUPS // 2 - 1, body, 0)

    drain_gather(buf0, sg0)
    fire_out(NGROUPS - 2, buf0, so0)
    drain_gather(buf1, sg1)
    fire_out(NGROUPS - 1, buf1, so1)
    drain_out(buf0, so0)
    drain_out(buf1, so1)


def kernel(entity_ids, table):
    ids = entity_ids.astype(jnp.int32).reshape(NW, CHUNKS_PER_W, CHUNK)
    flat = _sc_gather(ids, table)
    return flat.reshape(BATCH, HIST, EMBED_DIM)
